# Initial kernel scaffold; baseline (speedup 1.0000x reference)
import jax
import jax.numpy as jnp
from jax.experimental import pallas as pl


def kernel(x, y, z):
    raise NotImplementedError("write your pallas kernel here")


# speedup vs baseline: 6.8440x; 6.8440x over previous
import functools

import jax
import jax.numpy as jnp
from jax import lax
from jax.experimental import pallas as pl
from jax.experimental.pallas import tpu as pltpu
from jax.experimental.pallas import tpu_sc as plsc

NC = 2
NS = 16
L = 16
NW = NC * NS
ROWS = 128
COLS = 32768
RPW = ROWS // NW
SEG = 16384
HALVES = COLS // SEG
UNROLL = 8
TRIPS = SEG // (L * UNROLL)

_BIG_I32 = 2**31 - 1


def _sc_body(x_hbm, z_hbm, outx_hbm, outzv_hbm, outzi_hbm,
             xb0, xb1, zb0, zb1, ox, ozv, ozi, semx, semz):
  w = lax.axis_index("s") * NC + lax.axis_index("c")
  row0 = w * RPW

  iota = lax.iota(jnp.int32, L)
  negv = jnp.full((L,), -jnp.inf, jnp.float32)
  zeroi = jnp.zeros((L,), jnp.int32)

  stages = [(r, h) for r in range(RPW) for h in range(HALVES)]
  bufs = [(xb0, zb0), (xb1, zb1)]

  def dma_pair(s):
    r, h = stages[s]
    xb, zb = bufs[s % 2]
    cx = pltpu.async_copy(x_hbm.at[row0 + r, pl.ds(h * SEG, SEG)], xb, semx)
    cz = pltpu.async_copy(z_hbm.at[row0 + r, pl.ds(h * SEG, SEG)], zb, semz)
    return cx, cz

  def make_body(xb, zb, col0):
    def body(i, c):
      x0, x1, x2, x3, v0, v1, v2, i0, i1, i2 = c
      for u in range(UNROLL):
        off = (i * UNROLL + u) * L
        xv = xb[pl.ds(off, L)]
        cm = xv > x0
        t = jnp.where(cm, x0, xv); x0 = jnp.where(cm, xv, x0); xv = t
        cm = xv > x1
        t = jnp.where(cm, x1, xv); x1 = jnp.where(cm, xv, x1); xv = t
        cm = xv > x2
        t = jnp.where(cm, x2, xv); x2 = jnp.where(cm, xv, x2); xv = t
        x3 = jnp.maximum(x3, xv)
        zv = zb[pl.ds(off, L)]
        zx = iota + (col0 + off)
        cm = zv > v0
        tv = jnp.where(cm, v0, zv); ti = jnp.where(cm, i0, zx)
        v0 = jnp.where(cm, zv, v0); i0 = jnp.where(cm, zx, i0)
        zv, zx = tv, ti
        cm = zv > v1
        tv = jnp.where(cm, v1, zv); ti = jnp.where(cm, i1, zx)
        v1 = jnp.where(cm, zv, v1); i1 = jnp.where(cm, zx, i1)
        zv, zx = tv, ti
        cm = zv > v2
        v2 = jnp.where(cm, zv, v2); i2 = jnp.where(cm, zx, i2)
      return (x0, x1, x2, x3, v0, v1, v2, i0, i1, i2)
    return body

  cpair = dma_pair(0)
  carry = None
  for s, (r, h) in enumerate(stages):
    nxt = dma_pair(s + 1) if s + 1 < len(stages) else None
    cx, cz = cpair
    cx.wait()
    cz.wait()
    cpair = nxt
    xb, zb = bufs[s % 2]
    if h == 0:
      carry = (negv, negv, negv, negv, negv, negv, negv, zeroi, zeroi, zeroi)
    carry = lax.fori_loop(0, TRIPS, make_body(xb, zb, h * SEG), carry)
    if h == HALVES - 1:
      x0, x1, x2, x3, v0, v1, v2, i0, i1, i2 = carry
      resx = negv
      for j in range(4):
        mx = jnp.max(x0)
        resx = jnp.where(iota == j, mx, resx)
        eq = x0 == mx
        sel = eq & (iota == plsc.all_reduce_ffs(eq))
        x0 = jnp.where(sel, x1, x0)
        x1 = jnp.where(sel, x2, x1)
        x2 = jnp.where(sel, x3, x2)
        x3 = jnp.where(sel, negv, x3)
      ox[r] = resx
      reszv = negv
      reszi = zeroi
      for j in range(3):
        mz = jnp.max(v0)
        eq = v0 == mz
        mi = jnp.min(jnp.where(eq, i0, _BIG_I32))
        reszv = jnp.where(iota == j, mz, reszv)
        reszi = jnp.where(iota == j, mi, reszi)
        sel = eq & (i0 == mi)
        v0 = jnp.where(sel, v1, v0); i0 = jnp.where(sel, i1, i0)
        v1 = jnp.where(sel, v2, v1); i1 = jnp.where(sel, i2, i1)
        v2 = jnp.where(sel, negv, v2); i2 = jnp.where(sel, zeroi, i2)
      ozv[r] = reszv
      ozi[r] = reszi

  pltpu.sync_copy(ox, outx_hbm.at[pl.ds(row0, RPW)])
  pltpu.sync_copy(ozv, outzv_hbm.at[pl.ds(row0, RPW)])
  pltpu.sync_copy(ozi, outzi_hbm.at[pl.ds(row0, RPW)])


_sc_topk = functools.partial(
    pl.kernel,
    mesh=plsc.VectorSubcoreMesh(core_axis_name="c", subcore_axis_name="s"),
    compiler_params=pltpu.CompilerParams(needs_layout_passes=False),
    out_type=[
        jax.ShapeDtypeStruct((ROWS, 16), jnp.float32),
        jax.ShapeDtypeStruct((ROWS, 16), jnp.float32),
        jax.ShapeDtypeStruct((ROWS, 16), jnp.int32),
    ],
    scratch_types=[
        pltpu.VMEM((SEG,), jnp.float32),
        pltpu.VMEM((SEG,), jnp.float32),
        pltpu.VMEM((SEG,), jnp.float32),
        pltpu.VMEM((SEG,), jnp.float32),
        pltpu.VMEM((RPW, 16), jnp.float32),
        pltpu.VMEM((RPW, 16), jnp.float32),
        pltpu.VMEM((RPW, 16), jnp.int32),
        pltpu.SemaphoreType.DMA,
        pltpu.SemaphoreType.DMA,
    ],
)(_sc_body)


YB = 8


def _ymin_body(y_ref, o_ref):
  m = jnp.min(y_ref[...], axis=1)
  acc = m[:, 0:8]
  for i in range(1, 16):
    acc = jnp.minimum(acc, m[:, 8 * i:8 * i + 8])
  o_ref[...] = acc


_ymin = pl.pallas_call(
    _ymin_body,
    grid=(512 // YB,),
    in_specs=[pl.BlockSpec((YB, 256, 128), lambda i: (i, 0, 0))],
    out_specs=pl.BlockSpec((YB, 8), lambda i: (i, 0)),
    out_shape=jax.ShapeDtypeStruct((512, 8), jnp.float32),
)


def kernel(x, y, z):
  xo, zvo, zio = _sc_topk(x, z)
  ym = _ymin(y.reshape(512, 256, 128))
  return (xo[:, :4], ym.reshape(32, 16, 1, 8), zvo[:, :3], zio[:, :3])


# speedup vs baseline: 7.2593x; 1.0607x over previous
import functools

import jax
import jax.numpy as jnp
from jax import lax
from jax.experimental import pallas as pl
from jax.experimental.pallas import tpu as pltpu
from jax.experimental.pallas import tpu_sc as plsc

NC = 2
NS = 16
L = 16
NW = NC * NS
ROWS = 128
COLS = 32768
RPW = ROWS // NW
SEG = 16384
HALVES = COLS // SEG
UNROLL = 8
TRIPS = SEG // (L * UNROLL)

_BIG_I32 = 2**31 - 1


def _sc_body(x_hbm, z_hbm, outx_hbm, outzv_hbm, outzi_hbm,
             xb0, xb1, zb0, zb1, ox, ozv, ozi, semx, semz):
  w = lax.axis_index("s") * NC + lax.axis_index("c")
  row0 = w * RPW

  iota = lax.iota(jnp.int32, L)
  negv = jnp.full((L,), -jnp.inf, jnp.float32)
  zeroi = jnp.zeros((L,), jnp.int32)

  stages = [(r, h) for r in range(RPW) for h in range(HALVES)]
  bufs = [(xb0, zb0), (xb1, zb1)]

  def dma_pair(s):
    r, h = stages[s]
    xb, zb = bufs[s % 2]
    cx = pltpu.async_copy(x_hbm.at[row0 + r, pl.ds(h * SEG, SEG)], xb, semx)
    cz = pltpu.async_copy(z_hbm.at[row0 + r, pl.ds(h * SEG, SEG)], zb, semz)
    return cx, cz

  def make_body(xb, zb, col0):
    def body(i, c):
      x0, x1, x2, x3, v0, v1, v2, i0, i1, i2 = c
      for u in range(UNROLL):
        off = (i * UNROLL + u) * L
        xv = xb[pl.ds(off, L)]
        cm = xv > x0
        t = jnp.where(cm, x0, xv); x0 = jnp.where(cm, xv, x0); xv = t
        cm = xv > x1
        t = jnp.where(cm, x1, xv); x1 = jnp.where(cm, xv, x1); xv = t
        cm = xv > x2
        t = jnp.where(cm, x2, xv); x2 = jnp.where(cm, xv, x2); xv = t
        x3 = jnp.maximum(x3, xv)
        zv = zb[pl.ds(off, L)]
        zx = iota + (col0 + off)
        cm = zv > v0
        tv = jnp.where(cm, v0, zv); ti = jnp.where(cm, i0, zx)
        v0 = jnp.where(cm, zv, v0); i0 = jnp.where(cm, zx, i0)
        zv, zx = tv, ti
        cm = zv > v1
        tv = jnp.where(cm, v1, zv); ti = jnp.where(cm, i1, zx)
        v1 = jnp.where(cm, zv, v1); i1 = jnp.where(cm, zx, i1)
        zv, zx = tv, ti
        cm = zv > v2
        v2 = jnp.where(cm, zv, v2); i2 = jnp.where(cm, zx, i2)
      return (x0, x1, x2, x3, v0, v1, v2, i0, i1, i2)
    return body

  cpair = dma_pair(0)
  carry = None
  for s, (r, h) in enumerate(stages):
    nxt = dma_pair(s + 1) if s + 1 < len(stages) else None
    cx, cz = cpair
    cx.wait()
    cz.wait()
    cpair = nxt
    xb, zb = bufs[s % 2]
    if h == 0:
      carry = (negv, negv, negv, negv, negv, negv, negv, zeroi, zeroi, zeroi)
    carry = lax.fori_loop(0, TRIPS, make_body(xb, zb, h * SEG), carry)
    if h == HALVES - 1:
      x0, x1, x2, x3, v0, v1, v2, i0, i1, i2 = carry
      resx = negv
      for j in range(4):
        mx = jnp.max(x0)
        resx = jnp.where(iota == j, mx, resx)
        eq = x0 == mx
        sel = eq & (iota == plsc.all_reduce_ffs(eq))
        x0 = jnp.where(sel, x1, x0)
        x1 = jnp.where(sel, x2, x1)
        x2 = jnp.where(sel, x3, x2)
        x3 = jnp.where(sel, negv, x3)
      ox[r] = resx
      reszv = negv
      reszi = zeroi
      for j in range(3):
        mz = jnp.max(v0)
        eq = v0 == mz
        mi = jnp.min(jnp.where(eq, i0, _BIG_I32))
        reszv = jnp.where(iota == j, mz, reszv)
        reszi = jnp.where(iota == j, mi, reszi)
        sel = eq & (i0 == mi)
        v0 = jnp.where(sel, v1, v0); i0 = jnp.where(sel, i1, i0)
        v1 = jnp.where(sel, v2, v1); i1 = jnp.where(sel, i2, i1)
        v2 = jnp.where(sel, negv, v2); i2 = jnp.where(sel, zeroi, i2)
      ozv[r] = reszv
      ozi[r] = reszi

  pltpu.sync_copy(ox, outx_hbm.at[pl.ds(row0, RPW)])
  pltpu.sync_copy(ozv, outzv_hbm.at[pl.ds(row0, RPW)])
  pltpu.sync_copy(ozi, outzi_hbm.at[pl.ds(row0, RPW)])


_sc_topk = functools.partial(
    pl.kernel,
    mesh=plsc.VectorSubcoreMesh(core_axis_name="c", subcore_axis_name="s"),
    compiler_params=pltpu.CompilerParams(needs_layout_passes=False),
    out_type=[
        jax.ShapeDtypeStruct((ROWS, 16), jnp.float32),
        jax.ShapeDtypeStruct((ROWS, 16), jnp.float32),
        jax.ShapeDtypeStruct((ROWS, 16), jnp.int32),
    ],
    scratch_types=[
        pltpu.VMEM((SEG,), jnp.float32),
        pltpu.VMEM((SEG,), jnp.float32),
        pltpu.VMEM((SEG,), jnp.float32),
        pltpu.VMEM((SEG,), jnp.float32),
        pltpu.VMEM((RPW, 16), jnp.float32),
        pltpu.VMEM((RPW, 16), jnp.float32),
        pltpu.VMEM((RPW, 16), jnp.int32),
        pltpu.SemaphoreType.DMA,
        pltpu.SemaphoreType.DMA,
    ],
)(_sc_body)


YB = 8


def _ymin_body(y_ref, o_ref):
  o_ref[...] = jnp.min(y_ref[...], axis=1, keepdims=True)


_ymin = pl.pallas_call(
    _ymin_body,
    grid=(512,),
    in_specs=[pl.BlockSpec((1, 4096, 8), lambda i: (i, 0, 0))],
    out_specs=pl.BlockSpec((1, 1, 8), lambda i: (i, 0, 0)),
    out_shape=jax.ShapeDtypeStruct((512, 1, 8), jnp.float32),
)


def kernel(x, y, z):
  xo, zvo, zio = _sc_topk(x, z)
  ym = _ymin(y.reshape(512, 4096, 8))
  return (xo[:, :4], ym.reshape(32, 16, 1, 8), zvo[:, :3], zio[:, :3])


# speedup vs baseline: 59.3899x; 8.1812x over previous
import functools

import jax
import jax.numpy as jnp
from jax import lax
from jax.experimental import pallas as pl
from jax.experimental.pallas import tpu as pltpu
from jax.experimental.pallas import tpu_sc as plsc

NC = 2
NS = 16
L = 16
NW = NC * NS
ROWS = 128
COLS = 32768
RPW = ROWS // NW
SEG = 16384
HALVES = COLS // SEG
UNROLL = 8
TRIPS = SEG // (L * UNROLL)

_BIG_I32 = 2**31 - 1


def _sc_body(x_hbm, z_hbm, outx_hbm, outzv_hbm, outzi_hbm,
             xb0, xb1, zb0, zb1, ox, ozv, ozi, semx, semz):
  w = lax.axis_index("s") * NC + lax.axis_index("c")
  row0 = w * RPW

  iota = lax.iota(jnp.int32, L)
  negv = jnp.full((L,), -jnp.inf, jnp.float32)
  zeroi = jnp.zeros((L,), jnp.int32)

  stages = [(r, h) for r in range(RPW) for h in range(HALVES)]
  bufs = [(xb0, zb0), (xb1, zb1)]

  def dma_pair(s):
    r, h = stages[s]
    xb, zb = bufs[s % 2]
    cx = pltpu.async_copy(x_hbm.at[row0 + r, pl.ds(h * SEG, SEG)], xb, semx)
    cz = pltpu.async_copy(z_hbm.at[row0 + r, pl.ds(h * SEG, SEG)], zb, semz)
    return cx, cz

  def make_body(xb, zb, col0):
    def body(i, c):
      x0, x1, x2, x3, v0, v1, v2, i0, i1, i2 = c
      for u in range(UNROLL):
        off = (i * UNROLL + u) * L
        xv = xb[pl.ds(off, L)]
        cm = xv > x0
        t = jnp.where(cm, x0, xv); x0 = jnp.where(cm, xv, x0); xv = t
        cm = xv > x1
        t = jnp.where(cm, x1, xv); x1 = jnp.where(cm, xv, x1); xv = t
        cm = xv > x2
        t = jnp.where(cm, x2, xv); x2 = jnp.where(cm, xv, x2); xv = t
        x3 = jnp.maximum(x3, xv)
        zv = zb[pl.ds(off, L)]
        zx = iota + (col0 + off)
        cm = zv > v0
        tv = jnp.where(cm, v0, zv); ti = jnp.where(cm, i0, zx)
        v0 = jnp.where(cm, zv, v0); i0 = jnp.where(cm, zx, i0)
        zv, zx = tv, ti
        cm = zv > v1
        tv = jnp.where(cm, v1, zv); ti = jnp.where(cm, i1, zx)
        v1 = jnp.where(cm, zv, v1); i1 = jnp.where(cm, zx, i1)
        zv, zx = tv, ti
        cm = zv > v2
        v2 = jnp.where(cm, zv, v2); i2 = jnp.where(cm, zx, i2)
      return (x0, x1, x2, x3, v0, v1, v2, i0, i1, i2)
    return body

  cpair = dma_pair(0)
  carry = None
  for s, (r, h) in enumerate(stages):
    nxt = dma_pair(s + 1) if s + 1 < len(stages) else None
    cx, cz = cpair
    cx.wait()
    cz.wait()
    cpair = nxt
    xb, zb = bufs[s % 2]
    if h == 0:
      carry = (negv, negv, negv, negv, negv, negv, negv, zeroi, zeroi, zeroi)
    carry = lax.fori_loop(0, TRIPS, make_body(xb, zb, h * SEG), carry)
    if h == HALVES - 1:
      x0, x1, x2, x3, v0, v1, v2, i0, i1, i2 = carry
      resx = negv
      for j in range(4):
        mx = jnp.max(x0)
        resx = jnp.where(iota == j, mx, resx)
        eq = x0 == mx
        sel = eq & (iota == plsc.all_reduce_ffs(eq))
        x0 = jnp.where(sel, x1, x0)
        x1 = jnp.where(sel, x2, x1)
        x2 = jnp.where(sel, x3, x2)
        x3 = jnp.where(sel, negv, x3)
      ox[r] = resx
      reszv = negv
      reszi = zeroi
      for j in range(3):
        mz = jnp.max(v0)
        eq = v0 == mz
        mi = jnp.min(jnp.where(eq, i0, _BIG_I32))
        reszv = jnp.where(iota == j, mz, reszv)
        reszi = jnp.where(iota == j, mi, reszi)
        sel = eq & (i0 == mi)
        v0 = jnp.where(sel, v1, v0); i0 = jnp.where(sel, i1, i0)
        v1 = jnp.where(sel, v2, v1); i1 = jnp.where(sel, i2, i1)
        v2 = jnp.where(sel, negv, v2); i2 = jnp.where(sel, zeroi, i2)
      ozv[r] = reszv
      ozi[r] = reszi

  pltpu.sync_copy(ox, outx_hbm.at[pl.ds(row0, RPW)])
  pltpu.sync_copy(ozv, outzv_hbm.at[pl.ds(row0, RPW)])
  pltpu.sync_copy(ozi, outzi_hbm.at[pl.ds(row0, RPW)])


_sc_topk = functools.partial(
    pl.kernel,
    mesh=plsc.VectorSubcoreMesh(core_axis_name="c", subcore_axis_name="s"),
    compiler_params=pltpu.CompilerParams(needs_layout_passes=False),
    out_type=[
        jax.ShapeDtypeStruct((ROWS, 16), jnp.float32),
        jax.ShapeDtypeStruct((ROWS, 16), jnp.float32),
        jax.ShapeDtypeStruct((ROWS, 16), jnp.int32),
    ],
    scratch_types=[
        pltpu.VMEM((SEG,), jnp.float32),
        pltpu.VMEM((SEG,), jnp.float32),
        pltpu.VMEM((SEG,), jnp.float32),
        pltpu.VMEM((SEG,), jnp.float32),
        pltpu.VMEM((RPW, 16), jnp.float32),
        pltpu.VMEM((RPW, 16), jnp.float32),
        pltpu.VMEM((RPW, 16), jnp.int32),
        pltpu.SemaphoreType.DMA,
        pltpu.SemaphoreType.DMA,
    ],
)(_sc_body)


YB = 8


def _ymin_body(y_ref, o_ref):
  o_ref[...] = jnp.min(y_ref[...], axis=2)


_ymin = pl.pallas_call(
    _ymin_body,
    grid=(512 // YB,),
    in_specs=[pl.BlockSpec((YB, 8, 4096), lambda i: (i, 0, 0))],
    out_specs=pl.BlockSpec((YB, 8), lambda i: (i, 0)),
    out_shape=jax.ShapeDtypeStruct((512, 8), jnp.float32),
)


def kernel(x, y, z):
  xo, zvo, zio = _sc_topk(x, z)
  yt = jnp.moveaxis(y, 2, 3).reshape(512, 8, 4096)
  ym = _ymin(yt)
  return (xo[:, :4], ym.reshape(32, 16, 1, 8), zvo[:, :3], zio[:, :3])


# speedup vs baseline: 62.1929x; 1.0472x over previous
import functools

import jax
import jax.numpy as jnp
from jax import lax
from jax.experimental import pallas as pl
from jax.experimental.pallas import tpu as pltpu
from jax.experimental.pallas import tpu_sc as plsc

NC = 2
NS = 16
L = 16
NW = NC * NS
ROWS = 128
COLS = 32768
RPW = ROWS // NW
SEG = 16384
HALVES = COLS // SEG
UNROLL = 8
TRIPS = SEG // (L * UNROLL)

_BIG_I32 = 2**31 - 1


def _sc_body(x_hbm, z_hbm, outx_hbm, outzv_hbm, outzi_hbm,
             xb0, xb1, zb0, zb1, ox, ozv, ozi, semx, semz):
  w = lax.axis_index("s") * NC + lax.axis_index("c")
  row0 = w * RPW

  iota = lax.iota(jnp.int32, L)
  negv = jnp.full((L,), -jnp.inf, jnp.float32)
  zeroi = jnp.zeros((L,), jnp.int32)

  stages = [(r, h) for r in range(RPW) for h in range(HALVES)]
  bufs = [(xb0, zb0), (xb1, zb1)]

  def dma_pair(s):
    r, h = stages[s]
    xb, zb = bufs[s % 2]
    cx = pltpu.async_copy(x_hbm.at[row0 + r, pl.ds(h * SEG, SEG)], xb, semx)
    cz = pltpu.async_copy(z_hbm.at[row0 + r, pl.ds(h * SEG, SEG)], zb, semz)
    return cx, cz

  def make_body(xb, zb, col0):
    def body(i, c):
      x0, x1, x2, x3, v0, v1, v2, i0, i1, i2 = c
      for u in range(UNROLL):
        off = (i * UNROLL + u) * L
        xv = xb[pl.ds(off, L)]
        cm = xv > x0
        t = jnp.where(cm, x0, xv); x0 = jnp.where(cm, xv, x0); xv = t
        cm = xv > x1
        t = jnp.where(cm, x1, xv); x1 = jnp.where(cm, xv, x1); xv = t
        cm = xv > x2
        t = jnp.where(cm, x2, xv); x2 = jnp.where(cm, xv, x2); xv = t
        x3 = jnp.maximum(x3, xv)
        zv = zb[pl.ds(off, L)]
        zx = iota + (col0 + off)
        cm = zv > v0
        tv = jnp.where(cm, v0, zv); ti = jnp.where(cm, i0, zx)
        v0 = jnp.where(cm, zv, v0); i0 = jnp.where(cm, zx, i0)
        zv, zx = tv, ti
        cm = zv > v1
        tv = jnp.where(cm, v1, zv); ti = jnp.where(cm, i1, zx)
        v1 = jnp.where(cm, zv, v1); i1 = jnp.where(cm, zx, i1)
        zv, zx = tv, ti
        cm = zv > v2
        v2 = jnp.where(cm, zv, v2); i2 = jnp.where(cm, zx, i2)
      return (x0, x1, x2, x3, v0, v1, v2, i0, i1, i2)
    return body

  cpair = dma_pair(0)
  carry = None
  for s, (r, h) in enumerate(stages):
    nxt = dma_pair(s + 1) if s + 1 < len(stages) else None
    cx, cz = cpair
    cx.wait()
    cz.wait()
    cpair = nxt
    xb, zb = bufs[s % 2]
    if h == 0:
      carry = (negv, negv, negv, negv, negv, negv, negv, zeroi, zeroi, zeroi)
    carry = lax.fori_loop(0, TRIPS, make_body(xb, zb, h * SEG), carry)
    if h == HALVES - 1:
      x0, x1, x2, x3, v0, v1, v2, i0, i1, i2 = carry
      resx = negv
      for j in range(4):
        mx = jnp.max(x0)
        resx = jnp.where(iota == j, mx, resx)
        eq = x0 == mx
        sel = eq & (iota == plsc.all_reduce_ffs(eq))
        x0 = jnp.where(sel, x1, x0)
        x1 = jnp.where(sel, x2, x1)
        x2 = jnp.where(sel, x3, x2)
        x3 = jnp.where(sel, negv, x3)
      ox[r] = resx
      reszv = negv
      reszi = zeroi
      for j in range(3):
        mz = jnp.max(v0)
        eq = v0 == mz
        mi = jnp.min(jnp.where(eq, i0, _BIG_I32))
        reszv = jnp.where(iota == j, mz, reszv)
        reszi = jnp.where(iota == j, mi, reszi)
        sel = eq & (i0 == mi)
        v0 = jnp.where(sel, v1, v0); i0 = jnp.where(sel, i1, i0)
        v1 = jnp.where(sel, v2, v1); i1 = jnp.where(sel, i2, i1)
        v2 = jnp.where(sel, negv, v2); i2 = jnp.where(sel, zeroi, i2)
      ozv[r] = reszv
      ozi[r] = reszi

  pltpu.sync_copy(ox, outx_hbm.at[pl.ds(row0, RPW)])
  pltpu.sync_copy(ozv, outzv_hbm.at[pl.ds(row0, RPW)])
  pltpu.sync_copy(ozi, outzi_hbm.at[pl.ds(row0, RPW)])


_sc_topk = functools.partial(
    pl.kernel,
    mesh=plsc.VectorSubcoreMesh(core_axis_name="c", subcore_axis_name="s"),
    compiler_params=pltpu.CompilerParams(needs_layout_passes=False),
    out_type=[
        jax.ShapeDtypeStruct((ROWS, 16), jnp.float32),
        jax.ShapeDtypeStruct((ROWS, 16), jnp.float32),
        jax.ShapeDtypeStruct((ROWS, 16), jnp.int32),
    ],
    scratch_types=[
        pltpu.VMEM((SEG,), jnp.float32),
        pltpu.VMEM((SEG,), jnp.float32),
        pltpu.VMEM((SEG,), jnp.float32),
        pltpu.VMEM((SEG,), jnp.float32),
        pltpu.VMEM((RPW, 16), jnp.float32),
        pltpu.VMEM((RPW, 16), jnp.float32),
        pltpu.VMEM((RPW, 16), jnp.int32),
        pltpu.SemaphoreType.DMA,
        pltpu.SemaphoreType.DMA,
    ],
)(_sc_body)


YB = 16


def _ymin_body(y_ref, o_ref):
  o_ref[...] = jnp.min(y_ref[...], axis=2)


_ymin = pl.pallas_call(
    _ymin_body,
    grid=(512 // YB,),
    in_specs=[pl.BlockSpec((YB, 8, 4096), lambda i: (i, 0, 0))],
    out_specs=pl.BlockSpec((YB, 8), lambda i: (i, 0)),
    out_shape=jax.ShapeDtypeStruct((512, 8), jnp.float32),
)


def kernel(x, y, z):
  xo, zvo, zio = _sc_topk(x, z)
  yt = jnp.moveaxis(y, 2, 3).reshape(512, 8, 4096)
  ym = _ymin(yt)
  return (xo[:, :4], ym.reshape(32, 16, 1, 8), zvo[:, :3], zio[:, :3])
